# baseline (speedup 1.0000x reference)
import jax
import jax.numpy as jnp
from jax import lax
from jax.experimental import pallas as pl
from jax.experimental.pallas import tpu as pltpu
from jax.experimental.pallas import tpu_sc as plsc

B = 16384
BH = B // 2
E = 128
NC = 2
NS = 16
NW = NC * NS
BPW = BH // NW
CHUNK = 128
NCHUNK = BPW // CHUNK
LANES = 16
RBLK = 4096
NSTEP = BH // RBLK


def _sc_body(logits_hbm, aflat_hbm, bflat_hbm, cell_hbm, edge_hbm,
             rows_out, sel_out, a_out, b_out,
             cell_v, edge_v, fi_v, rows_v, sel_v, a_v, b_v, semr, seme,
             semw):
    wid = lax.axis_index("s") * NC + lax.axis_index("c")
    pltpu.sync_copy(cell_hbm.at[pl.ds(wid * NCHUNK, NCHUNK)], cell_v)
    pltpu.sync_copy(edge_hbm.at[pl.ds(wid * NCHUNK, NCHUNK)], edge_v)
    for j in range(NCHUNK):
        for k in range(CHUNK // LANES):
            c = cell_v[j, pl.ds(k * LANES, LANES)]
            e = edge_v[j, pl.ds(k * LANES, LANES)]
            fi_v[j, pl.ds(k * LANES, LANES)] = c * E + e
    base = wid * BPW
    row_copies = []
    for j in range(NCHUNK):
        row_copies.append(pltpu.async_copy(
            logits_hbm.at[cell_v.at[j]], rows_v.at[pl.ds(j * CHUNK, CHUNK)],
            semr))
    elem_copies = []
    for j in range(NCHUNK):
        elem_copies.append(pltpu.async_copy(
            aflat_hbm.at[fi_v.at[j]], a_v.at[pl.ds(j * CHUNK, CHUNK)], seme))
        elem_copies.append(pltpu.async_copy(
            bflat_hbm.at[fi_v.at[j]], b_v.at[pl.ds(j * CHUNK, CHUNK)], seme))
    write_copies = []
    for j in range(NCHUNK):
        row_copies[j].wait()
        for k in range(CHUNK // LANES):
            rl = lax.iota(jnp.int32, LANES) + (j * CHUNK + k * LANES)
            e = edge_v[j, pl.ds(k * LANES, LANES)]
            sel_v[pl.ds(j * CHUNK + k * LANES, LANES)] = plsc.load_gather(
                rows_v, [rl, e])
        write_copies.append(pltpu.async_copy(
            rows_v.at[pl.ds(j * CHUNK, CHUNK)],
            rows_out.at[pl.ds(base + j * CHUNK, CHUNK)], semw))
    for c in elem_copies:
        c.wait()
    pltpu.sync_copy(sel_v, sel_out.at[pl.ds(base, BPW)])
    pltpu.sync_copy(a_v, a_out.at[pl.ds(base, BPW)])
    pltpu.sync_copy(b_v, b_out.at[pl.ds(base, BPW)])
    for c in write_copies:
        c.wait()


_sc_gather = pl.kernel(
    _sc_body,
    out_type=(
        jax.ShapeDtypeStruct((BH, E), jnp.float32),
        jax.ShapeDtypeStruct((BH,), jnp.float32),
        jax.ShapeDtypeStruct((BH,), jnp.float32),
        jax.ShapeDtypeStruct((BH,), jnp.float32),
    ),
    mesh=plsc.VectorSubcoreMesh(core_axis_name="c", subcore_axis_name="s"),
    compiler_params=pltpu.CompilerParams(needs_layout_passes=False),
    scratch_types=[
        pltpu.VMEM((NCHUNK, CHUNK), jnp.int32),
        pltpu.VMEM((NCHUNK, CHUNK), jnp.int32),
        pltpu.VMEM((NCHUNK, CHUNK), jnp.int32),
        pltpu.VMEM((BPW, E), jnp.float32),
        pltpu.VMEM((BPW,), jnp.float32),
        pltpu.VMEM((BPW,), jnp.float32),
        pltpu.VMEM((BPW,), jnp.float32),
        pltpu.SemaphoreType.DMA,
        pltpu.SemaphoreType.DMA,
        pltpu.SemaphoreType.DMA,
    ],
)


_HALF_LOG_2PI = 0.9189385332046727
_LOG2E = 1.4426950408889634
_LN2 = 0.6931471805599453


def _exp(x):
    return jnp.exp2(x * _LOG2E)


def _log(x):
    return jnp.log2(x) * _LN2


def _lgamma(x):
    x4 = x + 4.0
    z = 1.0 / x4
    z2 = z * z
    series = z * (0.08333333333333333 +
                  z2 * (-0.002777777777777778 + z2 * 0.0007936507936507937))
    st = (x4 - 0.5) * _log(x4) - x4 + _HALF_LOG_2PI + series
    prod = x * (x + 1.0) * (x + 2.0) * (x + 3.0)
    return st - _log(prod)


def _tc_body(rows_ref, sel_ref, a_ref, b_ref, t_ref, o_ref, lse_s):
    g = pl.program_id(0)

    x3 = rows_ref[...].reshape(RBLK // E, E, E)
    m3 = jnp.max(x3, axis=2)
    s3 = jnp.sum(_exp(x3 - m3[:, :, None]), axis=2)
    lse_s[pl.ds(g * (RBLK // E), RBLK // E), :] = m3 + _log(s3)

    @pl.when(g == NSTEP - 1)
    def _finish():
        lse = lse_s[...]
        sel = sel_ref[...]
        a = a_ref[...]
        b = b_ref[...]
        t = t_ref[...]
        p = _exp(sel - lse)
        log_edge = _log(p + 1e-10)
        log_t = ((a - 1.0) * _log(t) + (b - 1.0) * _log(1.0 - t)
                 + _lgamma(a + b) - _lgamma(a) - _lgamma(b))
        o_ref[...] = log_edge + log_t


def _tc_call(rows, sel2, a2, b2, t2):
    vec_spec = pl.BlockSpec((BH // E, E), lambda g: (0, 0))
    return pl.pallas_call(
        _tc_body,
        grid=(NSTEP,),
        in_specs=[
            pl.BlockSpec((RBLK, E), lambda g: (g, 0)),
            vec_spec, vec_spec, vec_spec, vec_spec,
        ],
        out_specs=pl.BlockSpec((BH // E, E), lambda g: (0, 0)),
        out_shape=jax.ShapeDtypeStruct((BH // E, E), jnp.float32),
        scratch_shapes=[pltpu.VMEM((BH // E, E), jnp.float32)],
    )(rows, sel2, a2, b2, t2)


def kernel(edge_logits, alpha, beta, t, cell_idx, edge_idx):
    cell = cell_idx.astype(jnp.int32).reshape(B // CHUNK, CHUNK)
    edge = edge_idx.astype(jnp.int32).reshape(B // CHUNK, CHUNK)
    t2 = t.astype(jnp.float32).reshape(B // E, E)
    aflat = lax.optimization_barrier(alpha.reshape(-1))
    bflat = lax.optimization_barrier(beta.reshape(-1))
    nh = BH // CHUNK
    outs = []
    for h in range(2):
        rows, sel, a_g, b_g = _sc_gather(
            edge_logits, aflat, bflat,
            cell[h * nh:(h + 1) * nh], edge[h * nh:(h + 1) * nh])
        outs.append(_tc_call(
            rows, sel.reshape(BH // E, E), a_g.reshape(BH // E, E),
            b_g.reshape(BH // E, E),
            lax.slice_in_dim(t2, h * (BH // E), (h + 1) * (BH // E))))
    return jnp.concatenate([o.reshape(BH) for o in outs])


# speedup vs baseline: 1.1122x; 1.1122x over previous
import jax
import jax.numpy as jnp
from jax import lax
from jax.experimental import pallas as pl
from jax.experimental.pallas import tpu as pltpu
from jax.experimental.pallas import tpu_sc as plsc

B = 16384
E = 128
NCELL = 100000
NC = 2
NS = 16
NW = NC * NS
BPW = B // NW
CHUNK = 128
NCHUNK = BPW // CHUNK
LANES = 16
RBLK = 4096
NSTEP = B // RBLK


def _sc_body(logits_hbm, aflat_hbm, bflat_hbm, cell_hbm, edge_hbm,
             rows_out, sel_out, a_out, b_out,
             cell_v, edge_v, fi_v, rows_v, sel_v, a_v, b_v, semr, seme,
             semw):
    wid = lax.axis_index("s") * NC + lax.axis_index("c")
    pltpu.sync_copy(cell_hbm.at[pl.ds(wid * NCHUNK, NCHUNK)], cell_v)
    pltpu.sync_copy(edge_hbm.at[pl.ds(wid * NCHUNK, NCHUNK)], edge_v)
    for j in range(NCHUNK):
        for k in range(CHUNK // LANES):
            c = cell_v[j, pl.ds(k * LANES, LANES)]
            e = edge_v[j, pl.ds(k * LANES, LANES)]
            fi_v[j, pl.ds(k * LANES, LANES)] = c * E + e
    base = wid * BPW
    row_copies = []
    for j in range(NCHUNK):
        row_copies.append(pltpu.async_copy(
            logits_hbm.at[cell_v.at[j]], rows_v.at[pl.ds(j * CHUNK, CHUNK)],
            semr))
    elem_copies = []
    for j in range(NCHUNK):
        elem_copies.append(pltpu.async_copy(
            aflat_hbm.at[fi_v.at[j]], a_v.at[pl.ds(j * CHUNK, CHUNK)], seme))
        elem_copies.append(pltpu.async_copy(
            bflat_hbm.at[fi_v.at[j]], b_v.at[pl.ds(j * CHUNK, CHUNK)], seme))
    write_copies = []
    for j in range(NCHUNK):
        row_copies[j].wait()
        write_copies.append(pltpu.async_copy(
            rows_v.at[pl.ds(j * CHUNK, CHUNK)],
            rows_out.at[pl.ds(base + j * CHUNK, CHUNK)], semw))
        for k in range(CHUNK // LANES):
            rl = lax.iota(jnp.int32, LANES) + (j * CHUNK + k * LANES)
            e = edge_v[j, pl.ds(k * LANES, LANES)]
            sel_v[pl.ds(j * CHUNK + k * LANES, LANES)] = plsc.load_gather(
                rows_v, [rl, e])
    for c in elem_copies:
        c.wait()
    pltpu.sync_copy(sel_v, sel_out.at[pl.ds(base, BPW)])
    pltpu.sync_copy(a_v, a_out.at[pl.ds(base, BPW)])
    pltpu.sync_copy(b_v, b_out.at[pl.ds(base, BPW)])
    for c in write_copies:
        c.wait()


_sc_gather = pl.kernel(
    _sc_body,
    out_type=(
        jax.ShapeDtypeStruct((B, E), jnp.float32),
        jax.ShapeDtypeStruct((B,), jnp.float32),
        jax.ShapeDtypeStruct((B,), jnp.float32),
        jax.ShapeDtypeStruct((B,), jnp.float32),
    ),
    mesh=plsc.VectorSubcoreMesh(core_axis_name="c", subcore_axis_name="s"),
    compiler_params=pltpu.CompilerParams(needs_layout_passes=False),
    scratch_types=[
        pltpu.VMEM((NCHUNK, CHUNK), jnp.int32),
        pltpu.VMEM((NCHUNK, CHUNK), jnp.int32),
        pltpu.VMEM((NCHUNK, CHUNK), jnp.int32),
        pltpu.VMEM((BPW, E), jnp.float32),
        pltpu.VMEM((BPW,), jnp.float32),
        pltpu.VMEM((BPW,), jnp.float32),
        pltpu.VMEM((BPW,), jnp.float32),
        pltpu.SemaphoreType.DMA,
        pltpu.SemaphoreType.DMA,
        pltpu.SemaphoreType.DMA,
    ],
)


_HALF_LOG_2PI = 0.9189385332046727
_LOG2E = 1.4426950408889634
_LN2 = 0.6931471805599453


def _exp(x):
    return jnp.exp2(x * _LOG2E)


def _log(x):
    return jnp.log2(x) * _LN2


def _lgamma(x):
    x4 = x + 4.0
    z = 1.0 / x4
    z2 = z * z
    series = z * (0.08333333333333333 +
                  z2 * (-0.002777777777777778 + z2 * 0.0007936507936507937))
    st = (x4 - 0.5) * _log(x4) - x4 + _HALF_LOG_2PI + series
    prod = x * (x + 1.0) * (x + 2.0) * (x + 3.0)
    return st - _log(prod)


def _tc_body(rows_ref, sel_ref, a_ref, b_ref, t_ref, o_ref, lse_s):
    g = pl.program_id(0)

    x3 = rows_ref[...].reshape(RBLK // E, E, E)
    m3 = jnp.max(x3, axis=2)
    s3 = jnp.sum(_exp(x3 - m3[:, :, None]), axis=2)
    lse_s[pl.ds(g * (RBLK // E), RBLK // E), :] = m3 + _log(s3)

    @pl.when(g == NSTEP - 1)
    def _finish():
        lse = lse_s[...]
        sel = sel_ref[...]
        a = a_ref[...]
        b = b_ref[...]
        t = t_ref[...]
        p = _exp(sel - lse)
        log_edge = _log(p + 1e-10)
        log_t = ((a - 1.0) * _log(t) + (b - 1.0) * _log(1.0 - t)
                 + _lgamma(a + b) - _lgamma(a) - _lgamma(b))
        o_ref[...] = log_edge + log_t


def _tc_call(rows, sel2, a2, b2, t2):
    vec_spec = pl.BlockSpec((B // E, E), lambda g: (0, 0))
    return pl.pallas_call(
        _tc_body,
        grid=(NSTEP,),
        in_specs=[
            pl.BlockSpec((RBLK, E), lambda g: (g, 0)),
            vec_spec, vec_spec, vec_spec, vec_spec,
        ],
        out_specs=pl.BlockSpec((B // E, E), lambda g: (0, 0)),
        out_shape=jax.ShapeDtypeStruct((B // E, E), jnp.float32),
        scratch_shapes=[pltpu.VMEM((B // E, E), jnp.float32)],
    )(rows, sel2, a2, b2, t2)


def kernel(edge_logits, alpha, beta, t, cell_idx, edge_idx):
    cell = cell_idx.astype(jnp.int32).reshape(B // CHUNK, CHUNK)
    edge = edge_idx.astype(jnp.int32).reshape(B // CHUNK, CHUNK)
    aflat = lax.optimization_barrier(alpha.reshape(-1))
    bflat = lax.optimization_barrier(beta.reshape(-1))
    rows, sel, a_g, b_g = _sc_gather(edge_logits, aflat, bflat, cell, edge)
    out2 = _tc_call(rows, sel.reshape(B // E, E), a_g.reshape(B // E, E),
                    b_g.reshape(B // E, E),
                    t.astype(jnp.float32).reshape(B // E, E))
    return out2.reshape(B)
